# Initial kernel scaffold; baseline (speedup 1.0000x reference)
import jax
import jax.numpy as jnp
from jax.experimental import pallas as pl


def kernel(features, segment_ids, W1, gamma, beta, W2, b2):
    raise NotImplementedError("write your pallas kernel here")


# speedup vs baseline: 2.2773x; 2.2773x over previous
import functools

import jax
import jax.numpy as jnp
from jax import lax
from jax.experimental import pallas as pl
from jax.experimental.pallas import tpu as pltpu
from jax.experimental.pallas import tpu_sc as plsc

TOTAL = 32768
B = 16
D = 256
L = 16
NC = 2
NS = 16
NW = NC * NS
ROWS_PER_TILE = TOTAL // NW
CHUNK_ROWS = 128
CHUNK_ELEMS = CHUNK_ROWS * D
N_CHUNKS = ROWS_PER_TILE // CHUNK_ROWS
GROUPS_PER_CHUNK = CHUNK_ROWS // 16
ACC_ELEMS = B * D


def _sc_body(feat_hbm, ids_hbm, out_hbm, buf_v, ids_v, acc_v, sem0, sem1):
    wid = lax.axis_index("c") * NS + lax.axis_index("s")
    base_row = wid * ROWS_PER_TILE

    pltpu.sync_copy(ids_hbm.at[pl.ds(base_row, ROWS_PER_TILE)], ids_v)

    def zbody(i, carry):
        acc_v[pl.ds(i * L, L)] = jnp.zeros((L,), jnp.float32)
        return carry
    lax.fori_loop(0, ACC_ELEMS // L, zbody, 0)

    def fold_group(c, slot, g):
        iv = ids_v[pl.ds(c * CHUNK_ROWS + g * 16, 16)]
        s_first = iv[0]
        s_last = iv[15]
        boff = slot * CHUNK_ELEMS + g * 16 * D

        def uniform(carry):
            sbase = s_first * D
            for j in range(D // L):
                m = buf_v[pl.ds(boff + j * L, L)]
                for r in range(1, 16):
                    m = jnp.maximum(m, buf_v[pl.ds(boff + r * D + j * L, L)])
                cur = acc_v[pl.ds(sbase + j * L, L)]
                acc_v[pl.ds(sbase + j * L, L)] = jnp.maximum(cur, m)
            return carry

        def per_row(carry):
            for r in range(16):
                s_r = iv[r]
                sbase = s_r * D
                for j in range(D // L):
                    cur = acc_v[pl.ds(sbase + j * L, L)]
                    acc_v[pl.ds(sbase + j * L, L)] = jnp.maximum(
                        cur, buf_v[pl.ds(boff + r * D + j * L, L)])
            return carry

        lax.cond(s_first == s_last, uniform, per_row, 0)

    def start_chunk(c, slot, sem):
        src = feat_hbm.at[pl.ds((base_row + c * CHUNK_ROWS) * D, CHUNK_ELEMS)]
        dst = buf_v.at[pl.ds(slot * CHUNK_ELEMS, CHUNK_ELEMS)]
        pltpu.async_copy(src, dst, sem)

    def wait_chunk(slot, sem):
        src = feat_hbm.at[pl.ds(0, CHUNK_ELEMS)]
        dst = buf_v.at[pl.ds(slot * CHUNK_ELEMS, CHUNK_ELEMS)]
        pltpu.make_async_copy(src, dst, sem).wait()

    sems = (sem0, sem1)
    start_chunk(0, 0, sem0)
    start_chunk(1, 1, sem1)

    def pair_body(p, carry):
        for slot in range(2):
            c = p * 2 + slot
            wait_chunk(slot, sems[slot])

            def gbody(g, gc):
                fold_group(c, slot, g)
                return gc
            lax.fori_loop(0, GROUPS_PER_CHUNK, gbody, 0)

            @pl.when(p < N_CHUNKS // 2 - 1)
            def _start_next():
                start_chunk(c + 2, slot, sems[slot])
        return carry
    lax.fori_loop(0, N_CHUNKS // 2, pair_body, 0)

    pltpu.sync_copy(acc_v, out_hbm.at[wid])


_sc_segmax = functools.partial(
    pl.kernel,
    out_type=jax.ShapeDtypeStruct((NW, ACC_ELEMS), jnp.float32),
    mesh=plsc.VectorSubcoreMesh(
        core_axis_name="c", subcore_axis_name="s",
        num_cores=NC, num_subcores=NS),
    scratch_types=[
        pltpu.VMEM((2 * CHUNK_ELEMS,), jnp.float32),
        pltpu.VMEM((ROWS_PER_TILE,), jnp.int32),
        pltpu.VMEM((ACC_ELEMS,), jnp.float32),
        pltpu.SemaphoreType.DMA,
        pltpu.SemaphoreType.DMA,
    ],
)(_sc_body)


def _tc_body(part_ref, w1_ref, g_ref, bt_ref, w2_ref, b2_ref,
             pooled_ref, proj_ref):
    part = part_ref[...]
    pooled = jnp.max(part, axis=0)
    pooled_ref[...] = pooled
    h = lax.dot_general(pooled, w1_ref[...],
                        (((1,), (1,)), ((), ())),
                        preferred_element_type=jnp.float32)
    mean = jnp.mean(h, axis=0, keepdims=True)
    var = jnp.mean((h - mean) ** 2, axis=0, keepdims=True)
    hn = (h - mean) / jnp.sqrt(var + 1e-5) * g_ref[...] + bt_ref[...]
    hr = jnp.maximum(hn, 0.0)
    proj_ref[...] = lax.dot_general(hr, w2_ref[...],
                                    (((1,), (1,)), ((), ())),
                                    preferred_element_type=jnp.float32) \
        + b2_ref[...]


def _tc_mlp(part3, W1, gamma, beta, W2, b2):
    return pl.pallas_call(
        _tc_body,
        out_shape=[
            jax.ShapeDtypeStruct((B, D), jnp.float32),
            jax.ShapeDtypeStruct((B, 128), jnp.float32),
        ],
    )(part3, W1, gamma.reshape(1, D), beta.reshape(1, D),
      W2, b2.reshape(1, 128))


def kernel(features, segment_ids, W1, gamma, beta, W2, b2):
    ids32 = segment_ids.astype(jnp.int32)
    feat_flat = features.reshape(-1)
    partials = _sc_segmax(feat_flat, ids32)
    part3 = partials.reshape(NW, B, D)
    pooled, proj = _tc_mlp(part3, W1, gamma, beta, W2, b2)
    return (pooled, proj)


# speedup vs baseline: 3.7265x; 1.6363x over previous
import functools

import jax
import jax.numpy as jnp
from jax import lax
from jax.experimental import pallas as pl
from jax.experimental.pallas import tpu as pltpu
from jax.experimental.pallas import tpu_sc as plsc

TOTAL = 32768
B = 16
D = 256
L = 16
NC = 2
NS = 16
NW = NC * NS
ROWS_PER_TILE = TOTAL // NW
CHUNK_ROWS = 128
N_CHUNKS = ROWS_PER_TILE // CHUNK_ROWS
GROUPS_PER_CHUNK = CHUNK_ROWS // 16


def _sc_body(feat_hbm, ids_hbm, out_hbm, buf_v, ids_v, acc_v, sem0, sem1):
    wid = lax.axis_index("c") * NS + lax.axis_index("s")
    base_row = wid * ROWS_PER_TILE

    pltpu.sync_copy(ids_hbm.at[pl.ds(base_row, ROWS_PER_TILE)], ids_v)

    def zbody(i, carry):
        zv = jnp.zeros((L,), jnp.float32)
        for j in range(D // L):
            acc_v[i, pl.ds(j * L, L)] = zv
        return carry
    lax.fori_loop(0, B, zbody, 0)

    def fold_group(c, slot, g):
        iv = ids_v[pl.ds(c * CHUNK_ROWS + g * 16, 16)]
        s_first = iv[0]
        s_last = iv[15]
        brow = slot * CHUNK_ROWS + g * 16

        def uniform(carry):
            for j in range(D // L):
                m = buf_v[brow, pl.ds(j * L, L)]
                for r in range(1, 16):
                    m = jnp.maximum(m, buf_v[brow + r, pl.ds(j * L, L)])
                cur = acc_v[s_first, pl.ds(j * L, L)]
                acc_v[s_first, pl.ds(j * L, L)] = jnp.maximum(cur, m)
            return carry

        def per_row(carry):
            for r in range(16):
                s_r = iv[r]
                for j in range(D // L):
                    cur = acc_v[s_r, pl.ds(j * L, L)]
                    acc_v[s_r, pl.ds(j * L, L)] = jnp.maximum(
                        cur, buf_v[brow + r, pl.ds(j * L, L)])
            return carry

        lax.cond(s_first == s_last, uniform, per_row, 0)

    def start_chunk(c, slot, sem):
        src = feat_hbm.at[pl.ds(base_row + c * CHUNK_ROWS, CHUNK_ROWS)]
        dst = buf_v.at[pl.ds(slot * CHUNK_ROWS, CHUNK_ROWS)]
        pltpu.async_copy(src, dst, sem)

    def wait_chunk(slot, sem):
        src = feat_hbm.at[pl.ds(0, CHUNK_ROWS)]
        dst = buf_v.at[pl.ds(slot * CHUNK_ROWS, CHUNK_ROWS)]
        pltpu.make_async_copy(src, dst, sem).wait()

    sems = (sem0, sem1)
    start_chunk(0, 0, sem0)
    start_chunk(1, 1, sem1)

    def pair_body(p, carry):
        for slot in range(2):
            c = p * 2 + slot
            wait_chunk(slot, sems[slot])

            def gbody(g, gc):
                fold_group(c, slot, g)
                return gc
            lax.fori_loop(0, GROUPS_PER_CHUNK, gbody, 0)

            @pl.when(p < N_CHUNKS // 2 - 1)
            def _start_next():
                start_chunk(c + 2, slot, sems[slot])
        return carry
    lax.fori_loop(0, N_CHUNKS // 2, pair_body, 0)

    pltpu.sync_copy(acc_v, out_hbm.at[wid])


_sc_segmax = functools.partial(
    pl.kernel,
    out_type=jax.ShapeDtypeStruct((NW, B, D), jnp.float32),
    mesh=plsc.VectorSubcoreMesh(
        core_axis_name="c", subcore_axis_name="s",
        num_cores=NC, num_subcores=NS),
    scratch_types=[
        pltpu.VMEM((2 * CHUNK_ROWS, D), jnp.float32),
        pltpu.VMEM((ROWS_PER_TILE,), jnp.int32),
        pltpu.VMEM((B, D), jnp.float32),
        pltpu.SemaphoreType.DMA,
        pltpu.SemaphoreType.DMA,
    ],
    compiler_params=pltpu.CompilerParams(use_tc_tiling_on_sc=True),
)(_sc_body)


def _tc_body(part_ref, w1_ref, g_ref, bt_ref, w2_ref, b2_ref,
             pooled_ref, proj_ref):
    part = part_ref[...]
    pooled = jnp.max(part, axis=0)
    pooled_ref[...] = pooled
    h = lax.dot_general(pooled, w1_ref[...],
                        (((1,), (1,)), ((), ())),
                        preferred_element_type=jnp.float32)
    mean = jnp.mean(h, axis=0, keepdims=True)
    var = jnp.mean((h - mean) ** 2, axis=0, keepdims=True)
    hn = (h - mean) / jnp.sqrt(var + 1e-5) * g_ref[...] + bt_ref[...]
    hr = jnp.maximum(hn, 0.0)
    proj_ref[...] = lax.dot_general(hr, w2_ref[...],
                                    (((1,), (1,)), ((), ())),
                                    preferred_element_type=jnp.float32) \
        + b2_ref[...]


def _tc_mlp(part3, W1, gamma, beta, W2, b2):
    return pl.pallas_call(
        _tc_body,
        out_shape=[
            jax.ShapeDtypeStruct((B, D), jnp.float32),
            jax.ShapeDtypeStruct((B, 128), jnp.float32),
        ],
    )(part3, W1, gamma.reshape(1, D), beta.reshape(1, D),
      W2, b2.reshape(1, 128))


def kernel(features, segment_ids, W1, gamma, beta, W2, b2):
    ids32 = segment_ids.astype(jnp.int32)
    partials = _sc_segmax(features, ids32)
    pooled, proj = _tc_mlp(partials, W1, gamma, beta, W2, b2)
    return (pooled, proj)
